# baseline (speedup 1.0000x reference)
import functools

import jax
import jax.numpy as jnp
from jax import lax
from jax.experimental import pallas as pl
from jax.experimental.pallas import tpu as pltpu
from jax.experimental.pallas import tpu_sc as plsc

N_NODES = 10000
N_EDGES = 320000
D_IN = 128
D_H = 256
N_CLASSES = 32
N_GRAPHS = 128

NUM_CORES = 2
NUM_SUBCORES = 16
CHUNK = 128

DH = 128

KB1, OB1 = 20, 4
KB2, OB2 = 20, 8
E_PAD = 327680

N_PAD = 10112
ROWS_PER_TILE = N_PAD // NUM_SUBCORES


def _make_aggregate(nblocks, kb, dtype, tail, ncores=NUM_CORES, nb_by_core=None):
    mesh = plsc.VectorSubcoreMesh(core_axis_name="c", subcore_axis_name="s",
                                  num_cores=ncores)

    @functools.partial(
        pl.kernel,
        out_type=jax.ShapeDtypeStruct((ncores, N_PAD) + tail, dtype),
        mesh=mesh,
        scratch_types=[
            pltpu.VMEM((kb, CHUNK), jnp.int32),
            pltpu.VMEM((kb, CHUNK), jnp.int32),
            pltpu.VMEM((kb, CHUNK), jnp.int32),
            pltpu.VMEM((kb, CHUNK), jnp.int32),
            pltpu.VMEM((CHUNK,) + tail, dtype),
            pltpu.VMEM((CHUNK,) + tail, dtype),
            pltpu.VMEM_SHARED((N_PAD,) + tail, dtype),
            pltpu.SemaphoreType.DMA,
            pltpu.SemaphoreType.DMA,
            pltpu.SemaphoreType.DMA,
        ],
    )
    def agg(table, src5, dst5, zeros, out, sv0, dv0, sv1, dv1, r0, r1, acc,
            semA, semB, semI):
        c = lax.axis_index("c")
        s = lax.axis_index("s")
        rbase = s * ROWS_PER_TILE
        pltpu.sync_copy(zeros.at[pl.ds(rbase, ROWS_PER_TILE)],
                        acc.at[pl.ds(rbase, ROWS_PER_TILE)])
        plsc.subcore_barrier()
        pltpu.sync_copy(src5.at[c, s, 0], sv0)
        pltpu.sync_copy(dst5.at[c, s, 0], dv0)

        def process_block(ob, src_v, dst_v, nsv, ndv):
            @pl.when(ob + 1 < nblocks)
            def _pf():
                pltpu.async_copy(src5.at[c, s, ob + 1], nsv, semI)
                pltpu.async_copy(dst5.at[c, s, ob + 1], ndv, semI)

            pltpu.async_copy(table.at[src_v.at[0]], r0, semA)
            pltpu.async_copy(table.at[src_v.at[1]], r1, semB)

            def inner(p, carry2):
                j0 = 2 * p
                pltpu.make_async_copy(table.at[src_v.at[j0]], r0, semA).wait()
                pltpu.sync_copy(r0, acc.at[dst_v.at[j0]], add=True)
                pltpu.async_copy(table.at[src_v.at[j0 + 2]], r0, semA)
                pltpu.make_async_copy(table.at[src_v.at[j0 + 1]], r1, semB).wait()
                pltpu.sync_copy(r1, acc.at[dst_v.at[j0 + 1]], add=True)
                pltpu.async_copy(table.at[src_v.at[j0 + 3]], r1, semB)
                return carry2

            lax.fori_loop(0, kb // 2 - 1, inner, 0)
            j0 = kb - 2
            pltpu.make_async_copy(table.at[src_v.at[j0]], r0, semA).wait()
            pltpu.sync_copy(r0, acc.at[dst_v.at[j0]], add=True)
            pltpu.make_async_copy(table.at[src_v.at[j0 + 1]], r1, semB).wait()
            pltpu.sync_copy(r1, acc.at[dst_v.at[j0 + 1]], add=True)

            @pl.when(ob + 1 < nblocks)
            def _pfw():
                pltpu.make_async_copy(src5.at[c, s, ob + 1], nsv, semI).wait()
                pltpu.make_async_copy(dst5.at[c, s, ob + 1], ndv, semI).wait()

        if nb_by_core is None:
            def outer(q, carry):
                ob = 2 * q
                process_block(ob, sv0, dv0, sv1, dv1)
                process_block(ob + 1, sv1, dv1, sv0, dv0)
                return carry

            lax.fori_loop(0, nblocks // 2, outer, 0)
        else:
            nb = jnp.where(c == 0, nb_by_core[0], nb_by_core[1])
            for ob in range(nblocks):
                sets = (sv0, dv0, sv1, dv1) if ob % 2 == 0 else (sv1, dv1, sv0, dv0)

                @pl.when(ob < nb)
                def _blk(ob=ob, sets=sets):
                    process_block(ob, *sets)

        plsc.subcore_barrier()
        pltpu.sync_copy(acc.at[pl.ds(rbase, ROWS_PER_TILE)],
                        out.at[c, pl.ds(rbase, ROWS_PER_TILE)])

    return agg


_agg1 = _make_aggregate(OB1, KB1, jnp.float32, (DH,))
_agg2 = _make_aggregate(OB2, KB2, jnp.float32, (DH,))


def _mlp_body(a0_ref, a1_ref, h_ref, wa_ref, ba_ref, wb_ref, bb_ref,
              o_ref, o2_ref):
    z = h_ref[...] + a0_ref[...] + a1_ref[...]
    t = jnp.dot(z, wa_ref[...], preferred_element_type=jnp.float32) + ba_ref[...]
    t = jnp.maximum(t, 0.0)
    h1 = jnp.dot(t, wb_ref[...], preferred_element_type=jnp.float32) + bb_ref[...]
    o_ref[...] = h1
    dh2 = h1.shape[1] // 2
    o2_ref[...] = h1.reshape(h1.shape[0], 2, dh2).transpose(1, 0, 2)


def _mlp(a0, a1, h, wa, ba, wb, bb):
    n, d = h.shape
    dh = a0.shape[1]
    dmid = wa.shape[1]
    dout = wb.shape[1]
    blk = 2000
    grid = n // blk
    return pl.pallas_call(
        _mlp_body,
        grid=(grid,),
        in_specs=[
            pl.BlockSpec((blk, dh), lambda i: (i, 0)),
            pl.BlockSpec((blk, dh), lambda i: (i, 0)),
            pl.BlockSpec((blk, d), lambda i: (i, 0)),
            pl.BlockSpec((d, dmid), lambda i: (0, 0)),
            pl.BlockSpec((1, dmid), lambda i: (0, 0)),
            pl.BlockSpec((dmid, dout), lambda i: (0, 0)),
            pl.BlockSpec((1, dout), lambda i: (0, 0)),
        ],
        out_specs=[
            pl.BlockSpec((blk, dout), lambda i: (i, 0)),
            pl.BlockSpec((2, blk, dout // 2), lambda i: (0, i, 0)),
        ],
        out_shape=[
            jax.ShapeDtypeStruct((n, dout), jnp.float32),
            jax.ShapeDtypeStruct((2, n, dout // 2), jnp.float32),
        ],
    )(a0, a1, h, wa, ba.reshape(1, -1), wb, bb.reshape(1, -1))


def _mlp_pool_body(a0_ref, a1_ref, h_ref, wa_ref, ba_ref, wb_ref, bb_ref,
                   b_ref, w3_ref, b3_ref, o_ref, acc_ref):
    i = pl.program_id(0)
    z = h_ref[...] + jnp.concatenate([a0_ref[...], a1_ref[...]], axis=1)
    t = jnp.dot(z, wa_ref[...], preferred_element_type=jnp.float32) + ba_ref[...]
    t = jnp.maximum(t, 0.0)
    h2 = jnp.dot(t, wb_ref[...], preferred_element_type=jnp.float32) + bb_ref[...]

    @pl.when(i == 0)
    def _init():
        acc_ref[...] = jnp.zeros_like(acc_ref)

    blk = b_ref.shape[2]
    seg = lax.broadcasted_iota(jnp.int32, (N_GRAPHS, blk), 0)
    mask = (seg == b_ref[...].reshape(1, blk)).astype(jnp.float32)
    acc_ref[...] += jnp.dot(mask, h2, preferred_element_type=jnp.float32)

    @pl.when(i == pl.num_programs(0) - 1)
    def _fin():
        o_ref[...] = (jnp.dot(acc_ref[...], w3_ref[...],
                              preferred_element_type=jnp.float32) + b3_ref[...])


def _mlp_pool(a0, a1, h, wa, ba, wb, bb, batch3d, w3, b3):
    n, d = h.shape
    dh = a0.shape[1]
    dmid = wa.shape[1]
    dout = wb.shape[1]
    dcls = w3.shape[1]
    blk = 2000
    grid = n // blk
    return pl.pallas_call(
        _mlp_pool_body,
        grid=(grid,),
        in_specs=[
            pl.BlockSpec((blk, dh), lambda i: (i, 0)),
            pl.BlockSpec((blk, dh), lambda i: (i, 0)),
            pl.BlockSpec((blk, d), lambda i: (i, 0)),
            pl.BlockSpec((d, dmid), lambda i: (0, 0)),
            pl.BlockSpec((1, dmid), lambda i: (0, 0)),
            pl.BlockSpec((dmid, dout), lambda i: (0, 0)),
            pl.BlockSpec((1, dout), lambda i: (0, 0)),
            pl.BlockSpec((1, 1, blk), lambda i: (i, 0, 0)),
            pl.BlockSpec((dout, dcls), lambda i: (0, 0)),
            pl.BlockSpec((1, dcls), lambda i: (0, 0)),
        ],
        out_specs=pl.BlockSpec((N_GRAPHS, dcls), lambda i: (0, 0)),
        out_shape=jax.ShapeDtypeStruct((N_GRAPHS, dcls), jnp.float32),
        scratch_shapes=[pltpu.VMEM((N_GRAPHS, dout), jnp.float32)],
    )(a0, a1, h, wa, ba.reshape(1, -1), wb, bb.reshape(1, -1),
      batch3d, w3, b3.reshape(1, -1))


def kernel(x, edge_index, batch, W1a, b1a, W1b, b1b, W2a, b2a, W2b, b2b, W3, b3):
    src = edge_index[0].astype(jnp.int32)
    dst = edge_index[1].astype(jnp.int32)
    zeros = jnp.zeros((N_PAD, DH), jnp.float32)
    batch3d = batch.astype(jnp.int32).reshape(N_NODES // 2000, 1, 2000)

    pad = E_PAD - N_EDGES
    src_p = jnp.concatenate([src, jnp.zeros((pad,), jnp.int32)])
    dst_p = jnp.concatenate([dst, jnp.full((pad,), N_NODES, jnp.int32)])

    src4 = src_p.reshape(NUM_CORES, NUM_SUBCORES, OB1, KB1, CHUNK)
    dst4 = dst_p.reshape(NUM_CORES, NUM_SUBCORES, OB1, KB1, CHUNK)

    sbase = src_p.reshape(NUM_SUBCORES, OB2, KB2, CHUNK)
    src3 = jnp.stack([sbase, jnp.flip(sbase, axis=1) + N_NODES])
    dbase = dst_p.reshape(NUM_SUBCORES, OB2, KB2, CHUNK)
    dst3 = jnp.stack([dbase, jnp.flip(dbase, axis=1)])

    a = _agg1(x, src4, dst4, zeros)
    h1, h1split = _mlp(a[0, :N_NODES], a[1, :N_NODES], x, W1a, b1a, W1b, b1b)
    a2 = _agg2(h1split.reshape(2 * N_NODES, DH), src3, dst3, zeros)
    return _mlp_pool(a2[0, :N_NODES], a2[1, :N_NODES], h1, W2a, b2a, W2b, b2b,
                     batch3d, W3, b3)


# speedup vs baseline: 1.1216x; 1.1216x over previous
import functools

import jax
import jax.numpy as jnp
from jax import lax
from jax.experimental import pallas as pl
from jax.experimental.pallas import tpu as pltpu
from jax.experimental.pallas import tpu_sc as plsc

N_NODES = 10000
N_EDGES = 320000
D_IN = 128
D_H = 256
N_CLASSES = 32
N_GRAPHS = 128

NUM_CORES = 2
NUM_SUBCORES = 16
CHUNK = 128

DH = 128

KB1, OB1 = 20, 4
KB2, OB2 = 20, 8
E_PAD = 327680

N_PAD = 10112
ROWS_PER_TILE = N_PAD // NUM_SUBCORES


def _make_aggregate(nblocks, kb, dtype, tail):
    mesh = plsc.VectorSubcoreMesh(core_axis_name="c", subcore_axis_name="s")

    @functools.partial(
        pl.kernel,
        out_type=jax.ShapeDtypeStruct((NUM_CORES, N_PAD) + tail, dtype),
        mesh=mesh,
        scratch_types=[
            pltpu.VMEM((kb, CHUNK), jnp.int32),
            pltpu.VMEM((kb, CHUNK), jnp.int32),
            pltpu.VMEM((kb, CHUNK), jnp.int32),
            pltpu.VMEM((kb, CHUNK), jnp.int32),
            pltpu.VMEM((CHUNK,) + tail, dtype),
            pltpu.VMEM((CHUNK,) + tail, dtype),
            pltpu.VMEM_SHARED((N_PAD,) + tail, dtype),
            pltpu.SemaphoreType.DMA,
            pltpu.SemaphoreType.DMA,
            pltpu.SemaphoreType.DMA,
        ],
    )
    def agg(table, src5, dst5, zeros, out, sv0, dv0, sv1, dv1, r0, r1, acc,
            semA, semB, semI):
        c = lax.axis_index("c")
        s = lax.axis_index("s")
        rbase = s * ROWS_PER_TILE
        pltpu.sync_copy(zeros.at[pl.ds(rbase, ROWS_PER_TILE)],
                        acc.at[pl.ds(rbase, ROWS_PER_TILE)])
        plsc.subcore_barrier()
        pltpu.sync_copy(src5.at[c, s, 0], sv0)
        pltpu.sync_copy(dst5.at[c, s, 0], dv0)

        def process_block(ob, src_v, dst_v, nsv, ndv):
            @pl.when(ob + 1 < nblocks)
            def _pf():
                pltpu.async_copy(src5.at[c, s, ob + 1], nsv, semI)
                pltpu.async_copy(dst5.at[c, s, ob + 1], ndv, semI)

            pltpu.async_copy(table.at[src_v.at[0]], r0, semA)
            pltpu.async_copy(table.at[src_v.at[1]], r1, semB)

            def inner(p, carry2):
                j0 = 2 * p
                pltpu.make_async_copy(table.at[src_v.at[j0]], r0, semA).wait()
                pltpu.sync_copy(r0, acc.at[dst_v.at[j0]], add=True)
                pltpu.async_copy(table.at[src_v.at[j0 + 2]], r0, semA)
                pltpu.make_async_copy(table.at[src_v.at[j0 + 1]], r1, semB).wait()
                pltpu.sync_copy(r1, acc.at[dst_v.at[j0 + 1]], add=True)
                pltpu.async_copy(table.at[src_v.at[j0 + 3]], r1, semB)
                return carry2

            lax.fori_loop(0, kb // 2 - 1, inner, 0)
            j0 = kb - 2
            pltpu.make_async_copy(table.at[src_v.at[j0]], r0, semA).wait()
            pltpu.sync_copy(r0, acc.at[dst_v.at[j0]], add=True)
            pltpu.make_async_copy(table.at[src_v.at[j0 + 1]], r1, semB).wait()
            pltpu.sync_copy(r1, acc.at[dst_v.at[j0 + 1]], add=True)

            @pl.when(ob + 1 < nblocks)
            def _pfw():
                pltpu.make_async_copy(src5.at[c, s, ob + 1], nsv, semI).wait()
                pltpu.make_async_copy(dst5.at[c, s, ob + 1], ndv, semI).wait()

        def outer(q, carry):
            ob = 2 * q
            process_block(ob, sv0, dv0, sv1, dv1)
            process_block(ob + 1, sv1, dv1, sv0, dv0)
            return carry

        lax.fori_loop(0, nblocks // 2, outer, 0)
        plsc.subcore_barrier()
        pltpu.sync_copy(acc.at[pl.ds(rbase, ROWS_PER_TILE)],
                        out.at[c, pl.ds(rbase, ROWS_PER_TILE)])

    return agg


_agg1 = _make_aggregate(OB1, KB1, jnp.float32, (DH,))
_agg2 = _make_aggregate(OB2, KB2, jnp.float32, (DH,))


def _mlp_body(a0_ref, a1_ref, h_ref, wa_ref, ba_ref, wb_ref, bb_ref,
              o_ref, o2_ref):
    z = h_ref[...] + a0_ref[...] + a1_ref[...]
    t = jnp.dot(z, wa_ref[...], preferred_element_type=jnp.float32) + ba_ref[...]
    t = jnp.maximum(t, 0.0)
    h1 = jnp.dot(t, wb_ref[...], preferred_element_type=jnp.float32) + bb_ref[...]
    o_ref[...] = h1
    dh2 = h1.shape[1] // 2
    o2_ref[...] = h1.reshape(h1.shape[0], 2, dh2).transpose(1, 0, 2)


def _mlp(a0, a1, h, wa, ba, wb, bb):
    n, d = h.shape
    dh = a0.shape[1]
    dmid = wa.shape[1]
    dout = wb.shape[1]
    blk = 2000
    grid = n // blk
    return pl.pallas_call(
        _mlp_body,
        grid=(grid,),
        in_specs=[
            pl.BlockSpec((blk, dh), lambda i: (i, 0)),
            pl.BlockSpec((blk, dh), lambda i: (i, 0)),
            pl.BlockSpec((blk, d), lambda i: (i, 0)),
            pl.BlockSpec((d, dmid), lambda i: (0, 0)),
            pl.BlockSpec((1, dmid), lambda i: (0, 0)),
            pl.BlockSpec((dmid, dout), lambda i: (0, 0)),
            pl.BlockSpec((1, dout), lambda i: (0, 0)),
        ],
        out_specs=[
            pl.BlockSpec((blk, dout), lambda i: (i, 0)),
            pl.BlockSpec((2, blk, dout // 2), lambda i: (0, i, 0)),
        ],
        out_shape=[
            jax.ShapeDtypeStruct((n, dout), jnp.float32),
            jax.ShapeDtypeStruct((2, n, dout // 2), jnp.float32),
        ],
    )(a0, a1, h, wa, ba.reshape(1, -1), wb, bb.reshape(1, -1))


def _mlp_pool_body(a0_ref, a1_ref, h_ref, wa_ref, ba_ref, wb_ref, bb_ref,
                   b_ref, w3_ref, b3_ref, o_ref, acc_ref):
    i = pl.program_id(0)
    z = h_ref[...] + jnp.concatenate([a0_ref[...], a1_ref[...]], axis=1)
    t = jnp.dot(z, wa_ref[...], preferred_element_type=jnp.float32) + ba_ref[...]
    t = jnp.maximum(t, 0.0)
    h2 = jnp.dot(t, wb_ref[...], preferred_element_type=jnp.float32) + bb_ref[...]

    @pl.when(i == 0)
    def _init():
        acc_ref[...] = jnp.zeros_like(acc_ref)

    blk = b_ref.shape[2]
    seg = lax.broadcasted_iota(jnp.int32, (N_GRAPHS, blk), 0)
    mask = (seg == b_ref[...].reshape(1, blk)).astype(jnp.float32)
    acc_ref[...] += jnp.dot(mask, h2, preferred_element_type=jnp.float32)

    @pl.when(i == pl.num_programs(0) - 1)
    def _fin():
        o_ref[...] = (jnp.dot(acc_ref[...], w3_ref[...],
                              preferred_element_type=jnp.float32) + b3_ref[...])


def _mlp_pool(a0, a1, h, wa, ba, wb, bb, batch3d, w3, b3):
    n, d = h.shape
    dh = a0.shape[1]
    dmid = wa.shape[1]
    dout = wb.shape[1]
    dcls = w3.shape[1]
    blk = 2000
    grid = n // blk
    return pl.pallas_call(
        _mlp_pool_body,
        grid=(grid,),
        in_specs=[
            pl.BlockSpec((blk, dh), lambda i: (i, 0)),
            pl.BlockSpec((blk, dh), lambda i: (i, 0)),
            pl.BlockSpec((blk, d), lambda i: (i, 0)),
            pl.BlockSpec((d, dmid), lambda i: (0, 0)),
            pl.BlockSpec((1, dmid), lambda i: (0, 0)),
            pl.BlockSpec((dmid, dout), lambda i: (0, 0)),
            pl.BlockSpec((1, dout), lambda i: (0, 0)),
            pl.BlockSpec((1, 1, blk), lambda i: (i, 0, 0)),
            pl.BlockSpec((dout, dcls), lambda i: (0, 0)),
            pl.BlockSpec((1, dcls), lambda i: (0, 0)),
        ],
        out_specs=pl.BlockSpec((N_GRAPHS, dcls), lambda i: (0, 0)),
        out_shape=jax.ShapeDtypeStruct((N_GRAPHS, dcls), jnp.float32),
        scratch_shapes=[pltpu.VMEM((N_GRAPHS, dout), jnp.float32)],
    )(a0, a1, h, wa, ba.reshape(1, -1), wb, bb.reshape(1, -1),
      batch3d, w3, b3.reshape(1, -1))


def kernel(x, edge_index, batch, W1a, b1a, W1b, b1b, W2a, b2a, W2b, b2b, W3, b3):
    src = edge_index[0].astype(jnp.int32)
    dst = edge_index[1].astype(jnp.int32)
    zeros = jnp.zeros((N_PAD, DH), jnp.float32)
    batch3d = batch.astype(jnp.int32).reshape(N_NODES // 2000, 1, 2000)

    pad = E_PAD - N_EDGES
    src_p = jnp.concatenate([src, jnp.zeros((pad,), jnp.int32)])
    dst_p = jnp.concatenate([dst, jnp.full((pad,), N_NODES, jnp.int32)])

    src4 = src_p.reshape(NUM_CORES, NUM_SUBCORES, OB1, KB1, CHUNK)
    dst4 = dst_p.reshape(NUM_CORES, NUM_SUBCORES, OB1, KB1, CHUNK)

    sbase = src_p.reshape(NUM_SUBCORES, OB2, KB2, CHUNK)
    src3 = jnp.stack([sbase, sbase + N_NODES])
    dbase = dst_p.reshape(NUM_SUBCORES, OB2, KB2, CHUNK)
    dst3 = jnp.stack([dbase, dbase])

    a = _agg1(x, src4, dst4, zeros)
    h1, h1split = _mlp(a[0, :N_NODES], a[1, :N_NODES], x, W1a, b1a, W1b, b1b)
    a2 = _agg2(h1split.reshape(2 * N_NODES, DH), src3, dst3, zeros)
    return _mlp_pool(a2[0, :N_NODES], a2[1, :N_NODES], h1, W2a, b2a, W2b, b2b,
                     batch3d, W3, b3)
